# baseline (speedup 1.0000x reference)
import jax
import jax.numpy as jnp
from jax import lax
from jax.experimental import pallas as pl
from jax.experimental.pallas import tpu as pltpu
from jax.experimental.pallas import tpu_sc as plsc

_B = 16384
_DIM = 64
_MARGIN = 1.0
_NC = 2
_NS = 16
_NW = _NC * _NS
_PER_W = _B // _NW
_C = 64
_NCHUNK = _PER_W // _C
_EPS = 1e-12


def _rsqrt16(x):
    i = lax.bitcast_convert_type(x, jnp.int32)
    i = jnp.int32(0x5F3759DF) - (i >> 1)
    y = lax.bitcast_convert_type(i, jnp.float32)
    xh = 0.5 * x
    for _ in range(3):
        y = y * (1.5 - xh * y * y)
    return y


def _dot4(a, b):
    return jnp.sum(a[0] * b[0] + a[1] * b[1] + a[2] * b[2] + a[3] * b[3])


def _load4(ref, r):
    return [ref[r, pl.ds(16 * j, 16)] for j in range(4)]


def _load4bf(ref, r):
    out = []
    for j in range(2):
        a, b = plsc.unpack(ref[r, pl.ds(32 * j, 32)],
                           format=plsc.PackFormat.INTERLEAVED)
        out += [a, b]
    return out


def _transh_body(ph_hbm, pt_hbm, nh_hbm, nt_hbm, pr_hbm, nr_hbm,
                 ent_hbm, rn_hbm, out_hbm,
                 idxs_a, idxs_b,
                 pha, pta, nha, nta, pra, nra,
                 phb, ptb, nhb, ntb, prb, nrb,
                 dots, coef, out_v, sem_a, sem_b):
    wid = lax.axis_index("s") * _NC + lax.axis_index("c")
    base = wid * _PER_W
    iota = lax.iota(jnp.int32, 16)
    zeros = jnp.zeros((16,), jnp.float32)

    def load_idx(ch, idxs):
        off = base + ch * _C
        for q, src in enumerate((ph_hbm, pt_hbm, nh_hbm, nt_hbm,
                                 pr_hbm, nr_hbm)):
            pltpu.sync_copy(src.at[pl.ds(off, _C)], idxs.at[q])

    def issue(idxs, bufs, sem):
        bph, bpt, bnh, bnt, bpr, bnr = bufs
        def group(g, carry):
            vecs = [idxs[q, pl.ds(g * 16, 16)] for q in range(6)]
            for k in range(16):
                r = g * 16 + k
                for q, dst in enumerate((bph, bpt, bnh, bnt)):
                    pltpu.async_copy(ent_hbm.at[pl.ds(vecs[q][k], 1), :],
                                     dst.at[pl.ds(r, 1), :], sem)
                for q, dst in enumerate((bpr, bnr)):
                    pltpu.async_copy(rn_hbm.at[pl.ds(2 * vecs[4 + q][k], 2), :],
                                     dst.at[pl.ds(2 * r, 2), :], sem)
            return carry
        lax.fori_loop(0, _C // 16, group, 0)

    def drain(bufs, sem):
        bph, bpt, bnh, bnt, bpr, bnr = bufs
        for dst in (bph, bpt, bnh, bnt):
            pltpu.make_async_copy(ent_hbm.at[pl.ds(0, _C), :],
                                  dst, sem).wait()
        for dst in (bpr, bnr):
            pltpu.make_async_copy(rn_hbm.at[pl.ds(0, 2 * _C), :],
                                  dst, sem).wait()

    def compute(bufs, acc):
        bph, bpt, bnh, bnt, bpr, bnr = bufs

        def phase_a_group(g, carry):
            def phase_a_row(k, vecs):
                r = g * 16 + k
                m = iota == k
                out = []
                for (bh, bt, brn, s0) in ((bph, bpt, bpr, 0),
                                          (bnh, bnt, bnr, 6)):
                    h = _load4bf(bh, r)
                    t = _load4bf(bt, r)
                    rr_ = _load4(brn, 2 * r)
                    u = _load4(brn, 2 * r + 1)
                    for i, s in enumerate((_dot4(u, u), _dot4(h, h),
                                           _dot4(t, t), _dot4(rr_, rr_),
                                           _dot4(h, u), _dot4(t, u))):
                        out.append(jnp.where(m, jnp.full((16,), s),
                                             vecs[s0 + i]))
                return tuple(out)
            vecs = lax.fori_loop(0, 16, phase_a_row, (zeros,) * 12)
            for i in range(12):
                dots[i, pl.ds(g * 16, 16)] = vecs[i]
            return carry

        lax.fori_loop(0, _C // 16, phase_a_group, 0)

        def phase_b(g, carry):
            sl = pl.ds(g * 16, 16)
            for s0, c0 in ((0, 0), (6, 4)):
                nn = dots[s0 + 0, sl]
                hh = dots[s0 + 1, sl]
                tt = dots[s0 + 2, sl]
                rr_ = dots[s0 + 3, sl]
                hn = dots[s0 + 4, sl]
                tn = dots[s0 + 5, sl]
                inv_n = _rsqrt16(jnp.maximum(nn, _EPS))
                sq = nn * inv_n * inv_n
                a_h = hn * inv_n
                a_t = tn * inv_n
                php = hh - a_h * a_h * (2.0 - sq)
                ptp = tt - a_t * a_t * (2.0 - sq)
                inv_h = _rsqrt16(jnp.maximum(php, _EPS))
                inv_t = _rsqrt16(jnp.maximum(ptp, _EPS))
                inv_r = _rsqrt16(jnp.maximum(rr_, _EPS))
                gamma = inv_h * a_h * inv_n - inv_t * a_t * inv_n
                coef[c0 + 0, sl] = inv_h
                coef[c0 + 1, sl] = inv_t
                coef[c0 + 2, sl] = inv_r
                coef[c0 + 3, sl] = gamma
            return carry

        lax.fori_loop(0, _C // 16, phase_b, 0)

        def phase_c_row(r, a):
            g = r // 16
            k = r - g * 16
            m = iota == k
            mf = jnp.where(m, 1.0, 0.0)
            sl = pl.ds(g * 16, 16)
            sc = []
            for (bh, bt, brn, c0) in ((bph, bpt, bpr, 0),
                                      (bnh, bnt, bnr, 4)):
                inv_h = jnp.sum(coef[c0 + 0, sl] * mf)
                inv_t = jnp.sum(coef[c0 + 1, sl] * mf)
                inv_r = jnp.sum(coef[c0 + 2, sl] * mf)
                gamma = jnp.sum(coef[c0 + 3, sl] * mf)
                h = _load4bf(bh, r)
                t = _load4bf(bt, r)
                rr_ = _load4(brn, 2 * r)
                u = _load4(brn, 2 * r + 1)
                c = [jnp.abs(inv_h * h[j] - inv_t * t[j]
                             + inv_r * rr_[j] - gamma * u[j])
                     for j in range(4)]
                sc.append(jnp.sum(c[0] + c[1] + c[2] + c[3]))
            contrib = jnp.maximum(sc[0] - sc[1] + _MARGIN, 0.0)
            return a + jnp.where(m, jnp.full((16,), contrib), zeros)

        return lax.fori_loop(0, _C, phase_c_row, acc)

    bufs_a = (pha, pta, nha, nta, pra, nra)
    bufs_b = (phb, ptb, nhb, ntb, prb, nrb)

    load_idx(0, idxs_a)
    issue(idxs_a, bufs_a, sem_a)

    def chunk_pair(s, acc):
        c = 2 * s
        load_idx(c + 1, idxs_b)
        issue(idxs_b, bufs_b, sem_b)
        drain(bufs_a, sem_a)
        acc = compute(bufs_a, acc)

        @pl.when(c + 2 < _NCHUNK)
        def _():
            load_idx(c + 2, idxs_a)
            issue(idxs_a, bufs_a, sem_a)
        drain(bufs_b, sem_b)
        return compute(bufs_b, acc)

    acc = lax.fori_loop(0, _NCHUNK // 2, chunk_pair, zeros)
    out_v[...] = acc
    pltpu.sync_copy(out_v, out_hbm.at[wid])


@jax.jit
def _transh_sc(pos_h, pos_t, pos_r, neg_h, neg_t, neg_r,
               ent_emb, rel_emb, norm_vec):
    f32 = jnp.float32
    i32 = jnp.int32
    bf16 = jnp.bfloat16
    rn = jnp.stack([rel_emb, norm_vec], axis=1).reshape(-1, _DIM)
    ent_bf = ent_emb.astype(jnp.bfloat16)
    call = pl.kernel(
        _transh_body,
        out_type=jax.ShapeDtypeStruct((_NW, 16), f32),
        mesh=plsc.VectorSubcoreMesh(core_axis_name="c", subcore_axis_name="s"),
        compiler_params=pltpu.CompilerParams(
            needs_layout_passes=False, use_tc_tiling_on_sc=False),
        scratch_types=[
            pltpu.VMEM((6, _C), i32),
            pltpu.VMEM((6, _C), i32),
            pltpu.VMEM((_C, _DIM), bf16),
            pltpu.VMEM((_C, _DIM), bf16),
            pltpu.VMEM((_C, _DIM), bf16),
            pltpu.VMEM((_C, _DIM), bf16),
            pltpu.VMEM((2 * _C, _DIM), f32),
            pltpu.VMEM((2 * _C, _DIM), f32),
            pltpu.VMEM((_C, _DIM), bf16),
            pltpu.VMEM((_C, _DIM), bf16),
            pltpu.VMEM((_C, _DIM), bf16),
            pltpu.VMEM((_C, _DIM), bf16),
            pltpu.VMEM((2 * _C, _DIM), f32),
            pltpu.VMEM((2 * _C, _DIM), f32),
            pltpu.VMEM((12, _C), f32),
            pltpu.VMEM((8, _C), f32),
            pltpu.VMEM((16,), f32),
            pltpu.SemaphoreType.DMA,
            pltpu.SemaphoreType.DMA,
        ],
    )
    return call(pos_h, pos_t, neg_h, neg_t, pos_r, neg_r, ent_bf, rn)


def kernel(pos_h, pos_t, pos_r, neg_h, neg_t, neg_r,
           ent_emb, rel_emb, norm_vec):
    partials = _transh_sc(pos_h, pos_t, pos_r, neg_h, neg_t, neg_r,
                          ent_emb, rel_emb, norm_vec)
    return jnp.sum(partials) / _B


# speedup vs baseline: 2.1913x; 2.1913x over previous
import jax
import jax.numpy as jnp
from jax import lax
from jax.experimental import pallas as pl
from jax.experimental.pallas import tpu as pltpu
from jax.experimental.pallas import tpu_sc as plsc

_B = 16384
_DIM = 64
_MARGIN = 1.0
_NC = 2
_NS = 16
_NW = _NC * _NS
_PER_W = _B // _NW
_C = 64
_NCHUNK = _PER_W // _C
_EPS = 1e-12


def _rsqrt16(x):
    i = lax.bitcast_convert_type(x, jnp.int32)
    i = jnp.int32(0x5F3759DF) - (i >> 1)
    y = lax.bitcast_convert_type(i, jnp.float32)
    xh = 0.5 * x
    for _ in range(3):
        y = y * (1.5 - xh * y * y)
    return y


def _dot4(a, b):
    return jnp.sum(a[0] * b[0] + a[1] * b[1] + a[2] * b[2] + a[3] * b[3])


def _load4(ref, r, off=0):
    return [ref[r, pl.ds(off + 16 * j, 16)] for j in range(4)]


def _transh_body(ph_hbm, pt_hbm, nh_hbm, nt_hbm, pr_hbm, nr_hbm,
                 ent_hbm, rn_hbm, out_hbm,
                 idxs,
                 pha, pta, nha, nta, pra, nra,
                 phb, ptb, nhb, ntb, prb, nrb,
                 dots, coef, out_v, sem_a, sem_b):
    wid = lax.axis_index("s") * _NC + lax.axis_index("c")
    base = wid * _PER_W
    iota = lax.iota(jnp.int32, 16)
    zeros = jnp.zeros((16,), jnp.float32)

    def issue(ch, bufs, sem):
        bph, bpt, bnh, bnt, bpr, bnr = bufs
        def group(g, carry):
            vecs = [idxs[q, pl.ds(ch * _C + g * 16, 16)] for q in range(6)]
            for k in range(16):
                r = g * 16 + k
                for q, dst in enumerate((bph, bpt, bnh, bnt)):
                    pltpu.async_copy(ent_hbm.at[pl.ds(vecs[q][k], 1), :],
                                     dst.at[pl.ds(r, 1), :], sem)
                for q, dst in enumerate((bpr, bnr)):
                    pltpu.async_copy(rn_hbm.at[pl.ds(vecs[4 + q][k], 1), :],
                                     dst.at[pl.ds(r, 1), :], sem)
            return carry
        lax.fori_loop(0, _C // 16, group, 0)

    def drain(bufs, sem):
        bph, bpt, bnh, bnt, bpr, bnr = bufs
        for dst in (bph, bpt, bnh, bnt):
            pltpu.make_async_copy(ent_hbm.at[pl.ds(0, _C), :],
                                  dst, sem).wait()
        for dst in (bpr, bnr):
            pltpu.make_async_copy(rn_hbm.at[pl.ds(0, _C), :],
                                  dst, sem).wait()

    def compute(bufs, acc):
        bph, bpt, bnh, bnt, bpr, bnr = bufs

        def phase_a_group(g, carry):
            def phase_a_row(k, vecs):
                r = g * 16 + k
                m = iota == k
                out = []
                for (bh, bt, brn, s0) in ((bph, bpt, bpr, 0),
                                          (bnh, bnt, bnr, 6)):
                    h = _load4(bh, r)
                    t = _load4(bt, r)
                    rr_ = _load4(brn, r)
                    u = _load4(brn, r, 64)
                    for i, s in enumerate((_dot4(u, u), _dot4(h, h),
                                           _dot4(t, t), _dot4(rr_, rr_),
                                           _dot4(h, u), _dot4(t, u))):
                        out.append(jnp.where(m, jnp.full((16,), s),
                                             vecs[s0 + i]))
                return tuple(out)
            vecs = lax.fori_loop(0, 16, phase_a_row, (zeros,) * 12)
            for i in range(12):
                dots[i, pl.ds(g * 16, 16)] = vecs[i]
            return carry

        lax.fori_loop(0, _C // 16, phase_a_group, 0)

        def phase_b(g, carry):
            sl = pl.ds(g * 16, 16)
            for s0, c0 in ((0, 0), (6, 4)):
                nn = dots[s0 + 0, sl]
                hh = dots[s0 + 1, sl]
                tt = dots[s0 + 2, sl]
                rr_ = dots[s0 + 3, sl]
                hn = dots[s0 + 4, sl]
                tn = dots[s0 + 5, sl]
                inv_n = _rsqrt16(jnp.maximum(nn, _EPS))
                sq = nn * inv_n * inv_n
                a_h = hn * inv_n
                a_t = tn * inv_n
                php = hh - a_h * a_h * (2.0 - sq)
                ptp = tt - a_t * a_t * (2.0 - sq)
                inv_h = _rsqrt16(jnp.maximum(php, _EPS))
                inv_t = _rsqrt16(jnp.maximum(ptp, _EPS))
                inv_r = _rsqrt16(jnp.maximum(rr_, _EPS))
                gamma = inv_h * a_h * inv_n - inv_t * a_t * inv_n
                coef[c0 + 0, sl] = inv_h
                coef[c0 + 1, sl] = inv_t
                coef[c0 + 2, sl] = inv_r
                coef[c0 + 3, sl] = gamma
            return carry

        lax.fori_loop(0, _C // 16, phase_b, 0)

        def phase_c_row(r, a):
            g = r // 16
            k = r - g * 16
            m = iota == k
            mf = jnp.where(m, 1.0, 0.0)
            sl = pl.ds(g * 16, 16)
            sc = []
            for (bh, bt, brn, c0) in ((bph, bpt, bpr, 0),
                                      (bnh, bnt, bnr, 4)):
                inv_h = jnp.sum(coef[c0 + 0, sl] * mf)
                inv_t = jnp.sum(coef[c0 + 1, sl] * mf)
                inv_r = jnp.sum(coef[c0 + 2, sl] * mf)
                gamma = jnp.sum(coef[c0 + 3, sl] * mf)
                h = _load4(bh, r)
                t = _load4(bt, r)
                rr_ = _load4(brn, r)
                u = _load4(brn, r, 64)
                c = [jnp.abs(inv_h * h[j] - inv_t * t[j]
                             + inv_r * rr_[j] - gamma * u[j])
                     for j in range(4)]
                sc.append(jnp.sum(c[0] + c[1] + c[2] + c[3]))
            contrib = jnp.maximum(sc[0] - sc[1] + _MARGIN, 0.0)
            return a + jnp.where(m, jnp.full((16,), contrib), zeros)

        return lax.fori_loop(0, _C, phase_c_row, acc)

    bufs_a = (pha, pta, nha, nta, pra, nra)
    bufs_b = (phb, ptb, nhb, ntb, prb, nrb)

    for q, isrc in enumerate((ph_hbm, pt_hbm, nh_hbm, nt_hbm,
                              pr_hbm, nr_hbm)):
        pltpu.sync_copy(isrc.at[pl.ds(base, _PER_W)], idxs.at[q])

    issue(0, bufs_a, sem_a)

    def chunk_pair(s, acc):
        c = 2 * s
        issue(c + 1, bufs_b, sem_b)
        drain(bufs_a, sem_a)
        acc = compute(bufs_a, acc)

        @pl.when(c + 2 < _NCHUNK)
        def _():
            issue(c + 2, bufs_a, sem_a)
        drain(bufs_b, sem_b)
        return compute(bufs_b, acc)

    acc = lax.fori_loop(0, _NCHUNK // 2, chunk_pair, zeros)
    out_v[...] = acc
    pltpu.sync_copy(out_v, out_hbm.at[wid])


@jax.jit
def _transh_sc(pos_h, pos_t, pos_r, neg_h, neg_t, neg_r,
               ent_emb, rel_emb, norm_vec):
    f32 = jnp.float32
    i32 = jnp.int32
    rn = jnp.concatenate([rel_emb, norm_vec], axis=1)
    call = pl.kernel(
        _transh_body,
        out_type=jax.ShapeDtypeStruct((_NW, 16), f32),
        mesh=plsc.VectorSubcoreMesh(core_axis_name="c", subcore_axis_name="s"),
        compiler_params=pltpu.CompilerParams(
            needs_layout_passes=False, use_tc_tiling_on_sc=True),
        scratch_types=[
            pltpu.VMEM((6, _PER_W), i32),
            pltpu.VMEM((_C, _DIM), f32),
            pltpu.VMEM((_C, _DIM), f32),
            pltpu.VMEM((_C, _DIM), f32),
            pltpu.VMEM((_C, _DIM), f32),
            pltpu.VMEM((_C, 2 * _DIM), f32),
            pltpu.VMEM((_C, 2 * _DIM), f32),
            pltpu.VMEM((_C, _DIM), f32),
            pltpu.VMEM((_C, _DIM), f32),
            pltpu.VMEM((_C, _DIM), f32),
            pltpu.VMEM((_C, _DIM), f32),
            pltpu.VMEM((_C, 2 * _DIM), f32),
            pltpu.VMEM((_C, 2 * _DIM), f32),
            pltpu.VMEM((12, _C), f32),
            pltpu.VMEM((8, _C), f32),
            pltpu.VMEM((16,), f32),
            pltpu.SemaphoreType.DMA,
            pltpu.SemaphoreType.DMA,
        ],
    )
    return call(pos_h, pos_t, neg_h, neg_t, pos_r, neg_r, ent_emb, rn)


def kernel(pos_h, pos_t, pos_r, neg_h, neg_t, neg_r,
           ent_emb, rel_emb, norm_vec):
    partials = _transh_sc(pos_h, pos_t, pos_r, neg_h, neg_t, neg_r,
                          ent_emb, rel_emb, norm_vec)
    return jnp.sum(partials) / _B
